# baseline (speedup 1.0000x reference)
import functools

import jax
import jax.numpy as jnp
from jax import lax
from jax.experimental import pallas as pl
from jax.experimental.pallas import tpu as pltpu
from jax.experimental.pallas import tpu_sc as plsc

VOCAB = 1000000
EMBED_DIM = 32
BATCH = 16384
N_FIELDS = 26

_INFO = plsc.get_sparse_core_info()
_NC, _NS = _INFO.num_cores, _INFO.num_subcores
_NW = _NC * _NS

_B = BATCH * N_FIELDS
_BB = 128
_BLK_PER_W = BATCH // _BB // _NW
_IDX_PER_W = _BLK_PER_W * _BB * N_FIELDS
_UNITS = _BLK_PER_W * N_FIELDS


def _make_gather():
  mesh = plsc.VectorSubcoreMesh(core_axis_name="c", subcore_axis_name="s")

  @functools.partial(
      pl.kernel,
      mesh=mesh,
      out_type=jax.ShapeDtypeStruct((N_FIELDS, EMBED_DIM, BATCH), jnp.float32),
      scratch_types=[
          pltpu.VMEM((_IDX_PER_W,), jnp.int32),
          [pltpu.VMEM((_BB,), jnp.int32)] * 2,
          [pltpu.VMEM((_BB, EMBED_DIM), jnp.float32)] * 2,
          [pltpu.VMEM((EMBED_DIM, _BB), jnp.float32)] * 2,
          [pltpu.SemaphoreType.DMA] * 2,
          [pltpu.SemaphoreType.DMA] * 2,
      ],
      compiler_params=pltpu.CompilerParams(
          use_tc_tiling_on_sc=False, needs_layout_passes=False),
  )
  def gather_kernel(table_hbm, idx_hbm, out_hbm, idx_v, gl, rows, tbuf,
                    gsem, ssem):
    wid = lax.axis_index("s") * _NC + lax.axis_index("c")
    pltpu.sync_copy(idx_hbm.at[pl.ds(wid * _IDX_PER_W, _IDX_PER_W)], idx_v)

    iota = lax.iota(jnp.int32, 16)
    iota_nf = iota * N_FIELDS

    def unit_fb(u):
      bt = u // N_FIELDS
      f = u - bt * N_FIELDS
      return f, bt

    def build_and_fire(u, s):
      f, bt = unit_fb(u)
      ibase = bt * (_BB * N_FIELDS) + f
      for j in range(_BB // 16):
        v = plsc.load_gather(idx_v, [iota_nf + (ibase + j * 16 * N_FIELDS)])
        gl[s][pl.ds(j * 16, 16)] = v
      pltpu.async_copy(table_hbm.at[gl[s]], rows[s], gsem[s])

    def wait_gather(s):
      pltpu.make_async_copy(table_hbm.at[gl[s]], rows[s], gsem[s]).wait()

    def transpose_and_store(u, s):
      f, bt = unit_fb(u)
      for i in range(_BB):
        icol = iota * 0 + i
        for h in range(EMBED_DIM // 16):
          v = rows[s][i, pl.ds(h * 16, 16)]
          plsc.store_scatter(tbuf[s], [iota + h * 16, icol], v)
      b0 = (wid * _BLK_PER_W + bt) * _BB
      cp = pltpu.async_copy(
          tbuf[s], out_hbm.at[f, pl.ds(0, EMBED_DIM), pl.ds(b0, _BB)],
          ssem[s])
      cp.wait()

    build_and_fire(0, 0)
    build_and_fire(1, 1)

    @pl.loop(0, (_UNITS - 2) // 2)
    def _(t):
      for s in range(2):
        u = t * 2 + s
        wait_gather(s)
        transpose_and_store(u, s)
        build_and_fire(u + 2, s)

    for s in range(2):
      u = _UNITS - 2 + s
      wait_gather(s)
      transpose_and_store(u, s)

  return gather_kernel


_gather = _make_gather()


@jax.jit
def kernel(x, embedding_matrix):
  idx = x.reshape(_B).astype(jnp.int32)
  out = _gather(embedding_matrix, idx)
  return jnp.transpose(out, (2, 0, 1))


# speedup vs baseline: 1.1005x; 1.1005x over previous
import functools

import jax
import jax.numpy as jnp
from jax import lax
from jax.experimental import pallas as pl
from jax.experimental.pallas import tpu as pltpu
from jax.experimental.pallas import tpu_sc as plsc

VOCAB = 1000000
EMBED_DIM = 32
BATCH = 16384
N_FIELDS = 26

_INFO = plsc.get_sparse_core_info()
_NC, _NS = _INFO.num_cores, _INFO.num_subcores
_NW = _NC * _NS

_B = BATCH * N_FIELDS
_BB = 128
_BLK_PER_W = BATCH // _BB // _NW
_IDX_PER_W = _BLK_PER_W * _BB * N_FIELDS
_UNITS = _BLK_PER_W * N_FIELDS


def _make_gather():
  mesh = plsc.VectorSubcoreMesh(core_axis_name="c", subcore_axis_name="s")

  @functools.partial(
      pl.kernel,
      mesh=mesh,
      out_type=jax.ShapeDtypeStruct((N_FIELDS, EMBED_DIM, BATCH), jnp.float32),
      scratch_types=[
          pltpu.VMEM((_IDX_PER_W,), jnp.int32),
          [pltpu.VMEM((_BB,), jnp.int32)] * 2,
          [pltpu.VMEM((_BB, EMBED_DIM), jnp.float32)] * 2,
          [pltpu.VMEM((EMBED_DIM, _BB), jnp.float32)] * 2,
          [pltpu.SemaphoreType.DMA] * 2,
          [pltpu.SemaphoreType.DMA] * 2,
      ],
      compiler_params=pltpu.CompilerParams(
          use_tc_tiling_on_sc=False, needs_layout_passes=False),
  )
  def gather_kernel(table_hbm, idx_hbm, out_hbm, idx_v, gl, rows, tbuf,
                    gsem, ssem):
    wid = lax.axis_index("s") * _NC + lax.axis_index("c")
    pltpu.sync_copy(idx_hbm.at[pl.ds(wid * _IDX_PER_W, _IDX_PER_W)], idx_v)

    iota = lax.iota(jnp.int32, 16)
    iota_nf = iota * N_FIELDS

    def unit_fb(u):
      bt = u // N_FIELDS
      f = u - bt * N_FIELDS
      return f, bt

    def build_and_fire(u, s):
      f, bt = unit_fb(u)
      ibase = bt * (_BB * N_FIELDS) + f
      for j in range(_BB // 16):
        v = plsc.load_gather(idx_v, [iota_nf + (ibase + j * 16 * N_FIELDS)])
        gl[s][pl.ds(j * 16, 16)] = v
      pltpu.async_copy(table_hbm.at[gl[s]], rows[s], gsem[s])

    def wait_gather(s):
      pltpu.make_async_copy(table_hbm.at[gl[s]], rows[s], gsem[s]).wait()

    def transpose_and_store(u, s):
      f, bt = unit_fb(u)

      @plsc.parallel_loop(0, _BB, step=1, unroll=8)
      def _(i):
        icol = iota * 0 + i
        for h in range(EMBED_DIM // 16):
          v = rows[s][i, pl.ds(h * 16, 16)]
          plsc.store_scatter(tbuf[s], [iota + h * 16, icol], v)

      b0 = (wid * _BLK_PER_W + bt) * _BB
      cp = pltpu.async_copy(
          tbuf[s], out_hbm.at[f, pl.ds(0, EMBED_DIM), pl.ds(b0, _BB)],
          ssem[s])
      cp.wait()

    build_and_fire(0, 0)
    build_and_fire(1, 1)

    @pl.loop(0, (_UNITS - 2) // 2)
    def _(t):
      for s in range(2):
        u = t * 2 + s
        wait_gather(s)
        transpose_and_store(u, s)
        build_and_fire(u + 2, s)

    for s in range(2):
      u = _UNITS - 2 + s
      wait_gather(s)
      transpose_and_store(u, s)

  return gather_kernel


_gather = _make_gather()


@jax.jit
def kernel(x, embedding_matrix):
  idx = x.reshape(_B).astype(jnp.int32)
  out = _gather(embedding_matrix, idx)
  return jnp.transpose(out, (2, 0, 1))


# speedup vs baseline: 1.1036x; 1.0028x over previous
import functools

import jax
import jax.numpy as jnp
from jax import lax
from jax.experimental import pallas as pl
from jax.experimental.pallas import tpu as pltpu
from jax.experimental.pallas import tpu_sc as plsc

VOCAB = 1000000
EMBED_DIM = 32
BATCH = 16384
N_FIELDS = 26

_INFO = plsc.get_sparse_core_info()
_NC, _NS = _INFO.num_cores, _INFO.num_subcores
_NW = _NC * _NS

_B = BATCH * N_FIELDS
_BB = 128
_BLK_PER_W = BATCH // _BB // _NW
_IDX_PER_W = _BLK_PER_W * _BB * N_FIELDS
_UNITS = _BLK_PER_W * N_FIELDS


def _make_gather():
  mesh = plsc.VectorSubcoreMesh(core_axis_name="c", subcore_axis_name="s")

  @functools.partial(
      pl.kernel,
      mesh=mesh,
      out_type=jax.ShapeDtypeStruct((N_FIELDS, EMBED_DIM, BATCH), jnp.float32),
      scratch_types=[
          pltpu.VMEM((_IDX_PER_W,), jnp.int32),
          [pltpu.VMEM((_BB,), jnp.int32)] * 2,
          [pltpu.VMEM((_BB, EMBED_DIM), jnp.float32)] * 2,
          [pltpu.VMEM((EMBED_DIM, _BB), jnp.float32)] * 2,
          [pltpu.SemaphoreType.DMA] * 2,
          [pltpu.SemaphoreType.DMA] * 2,
      ],
      compiler_params=pltpu.CompilerParams(
          use_tc_tiling_on_sc=False, needs_layout_passes=False),
  )
  def gather_kernel(table_hbm, idx_hbm, out_hbm, idx_v, gl, rows, tbuf,
                    gsem, ssem):
    wid = lax.axis_index("s") * _NC + lax.axis_index("c")
    pltpu.sync_copy(idx_hbm.at[pl.ds(wid * _IDX_PER_W, _IDX_PER_W)], idx_v)

    iota = lax.iota(jnp.int32, 16)
    iota_nf = iota * N_FIELDS

    def unit_fb(u):
      bt = u // N_FIELDS
      f = u - bt * N_FIELDS
      return f, bt

    def build_and_fire(u, s):
      f, bt = unit_fb(u)
      ibase = bt * (_BB * N_FIELDS) + f
      for j in range(_BB // 16):
        v = plsc.load_gather(idx_v, [iota_nf + (ibase + j * 16 * N_FIELDS)])
        gl[s][pl.ds(j * 16, 16)] = v
      pltpu.async_copy(table_hbm.at[gl[s]], rows[s], gsem[s])

    def wait_gather(s):
      pltpu.make_async_copy(table_hbm.at[gl[s]], rows[s], gsem[s]).wait()

    def transpose_and_store(u, s):
      f, bt = unit_fb(u)

      @plsc.parallel_loop(0, _BB, step=1, unroll=16)
      def _(i):
        icol = iota * 0 + i
        for h in range(EMBED_DIM // 16):
          v = rows[s][i, pl.ds(h * 16, 16)]
          plsc.store_scatter(tbuf[s], [iota + h * 16, icol], v)

      b0 = (wid * _BLK_PER_W + bt) * _BB
      return pltpu.async_copy(
          tbuf[s], out_hbm.at[f, pl.ds(0, EMBED_DIM), pl.ds(b0, _BB)],
          ssem[s])

    build_and_fire(0, 0)
    build_and_fire(1, 1)

    @pl.loop(0, (_UNITS - 2) // 2)
    def _(t):
      for s in range(2):
        u = t * 2 + s
        wait_gather(s)
        cp = transpose_and_store(u, s)
        build_and_fire(u + 2, s)
        cp.wait()

    for s in range(2):
      u = _UNITS - 2 + s
      wait_gather(s)
      transpose_and_store(u, s).wait()

  return gather_kernel


_gather = _make_gather()


@jax.jit
def kernel(x, embedding_matrix):
  idx = x.reshape(_B).astype(jnp.int32)
  out = _gather(embedding_matrix, idx)
  return jnp.transpose(out, (2, 0, 1))


# speedup vs baseline: 1.1069x; 1.0030x over previous
import functools

import jax
import jax.numpy as jnp
from jax import lax
from jax.experimental import pallas as pl
from jax.experimental.pallas import tpu as pltpu
from jax.experimental.pallas import tpu_sc as plsc

VOCAB = 1000000
EMBED_DIM = 32
BATCH = 16384
N_FIELDS = 26

_INFO = plsc.get_sparse_core_info()
_NC, _NS = _INFO.num_cores, _INFO.num_subcores
_NW = _NC * _NS

_B = BATCH * N_FIELDS
_BB = 256
_BLK_PER_W = BATCH // _BB // _NW
_IDX_PER_W = _BLK_PER_W * _BB * N_FIELDS
_UNITS = _BLK_PER_W * N_FIELDS


def _make_gather():
  mesh = plsc.VectorSubcoreMesh(core_axis_name="c", subcore_axis_name="s")

  @functools.partial(
      pl.kernel,
      mesh=mesh,
      out_type=jax.ShapeDtypeStruct((N_FIELDS, EMBED_DIM, BATCH), jnp.float32),
      scratch_types=[
          pltpu.VMEM((_IDX_PER_W,), jnp.int32),
          [pltpu.VMEM((_BB,), jnp.int32)] * 2,
          [pltpu.VMEM((_BB, EMBED_DIM), jnp.float32)] * 2,
          [pltpu.VMEM((EMBED_DIM, _BB), jnp.float32)] * 2,
          [pltpu.SemaphoreType.DMA] * 2,
          [pltpu.SemaphoreType.DMA] * 2,
      ],
      compiler_params=pltpu.CompilerParams(
          use_tc_tiling_on_sc=False, needs_layout_passes=False),
  )
  def gather_kernel(table_hbm, idx_hbm, out_hbm, idx_v, gl, rows, tbuf,
                    gsem, ssem):
    wid = lax.axis_index("s") * _NC + lax.axis_index("c")
    pltpu.sync_copy(idx_hbm.at[pl.ds(wid * _IDX_PER_W, _IDX_PER_W)], idx_v)

    iota = lax.iota(jnp.int32, 16)
    iota_nf = iota * N_FIELDS

    def unit_fb(u):
      bt = u // N_FIELDS
      f = u - bt * N_FIELDS
      return f, bt

    def build_and_fire(u, s):
      f, bt = unit_fb(u)
      ibase = bt * (_BB * N_FIELDS) + f
      for j in range(_BB // 16):
        v = plsc.load_gather(idx_v, [iota_nf + (ibase + j * 16 * N_FIELDS)])
        gl[s][pl.ds(j * 16, 16)] = v
      pltpu.async_copy(table_hbm.at[gl[s]], rows[s], gsem[s])

    def wait_gather(s):
      pltpu.make_async_copy(table_hbm.at[gl[s]], rows[s], gsem[s]).wait()

    def transpose_and_store(u, s):
      f, bt = unit_fb(u)

      @plsc.parallel_loop(0, _BB, step=1, unroll=16)
      def _(i):
        icol = iota * 0 + i
        for h in range(EMBED_DIM // 16):
          v = rows[s][i, pl.ds(h * 16, 16)]
          plsc.store_scatter(tbuf[s], [iota + h * 16, icol], v)

      b0 = (wid * _BLK_PER_W + bt) * _BB
      return pltpu.async_copy(
          tbuf[s], out_hbm.at[f, pl.ds(0, EMBED_DIM), pl.ds(b0, _BB)],
          ssem[s])

    build_and_fire(0, 0)
    build_and_fire(1, 1)

    @pl.loop(0, (_UNITS - 2) // 2)
    def _(t):
      for s in range(2):
        u = t * 2 + s
        wait_gather(s)
        cp = transpose_and_store(u, s)
        build_and_fire(u + 2, s)
        cp.wait()

    for s in range(2):
      u = _UNITS - 2 + s
      wait_gather(s)
      transpose_and_store(u, s).wait()

  return gather_kernel


_gather = _make_gather()


@jax.jit
def kernel(x, embedding_matrix):
  idx = x.reshape(_B).astype(jnp.int32)
  out = _gather(embedding_matrix, idx)
  return jnp.transpose(out, (2, 0, 1))
